# baseline (speedup 1.0000x reference)
import dataclasses
import functools

import jax
import jax.numpy as jnp
from jax import lax
from jax.experimental import pallas as pl
from jax.experimental.pallas import tpu as pltpu
from jax.experimental.pallas import tpu_sc as plsc

NCLS = 17
FDIM = 64
NLANE = 16
NCORE = 2
NSUB = 16
NWORK = NCORE * NSUB
VB = 512
ACC_STRIDE = NCLS * FDIM + 1
CNT_STRIDE = 33
CNT_WORDS = NLANE * 34
SC_N = 98304


def _newton_rsqrt(x):
    i = lax.bitcast_convert_type(x, jnp.int32)
    y = lax.bitcast_convert_type(jnp.int32(0x5F3759DF) - (i >> 1), jnp.float32)
    for _ in range(3):
        y = y * (1.5 - 0.5 * x * y * y)
    return y


def _sc_segment_stats(emb, lbl, sc_n):
    batch = emb.shape[0]
    vox_per_worker = batch * sc_n // NWORK
    workers_per_batch = NWORK // batch
    n_blocks = vox_per_worker // VB
    assert n_blocks >= 2 and n_blocks % 2 == 0 and n_blocks * VB == vox_per_worker

    mesh = plsc.VectorSubcoreMesh(core_axis_name="c", subcore_axis_name="s")
    cp = pltpu.CompilerParams()
    if "needs_layout_passes" in pltpu.CompilerParams.__dataclass_fields__:
        cp = dataclasses.replace(cp, needs_layout_passes=False)

    @functools.partial(
        pl.kernel,
        compiler_params=cp,
        out_type=(
            jax.ShapeDtypeStruct((NWORK, NLANE * ACC_STRIDE), jnp.float32),
            jax.ShapeDtypeStruct((NWORK, CNT_WORDS), jnp.float32),
        ),
        mesh=mesh,
        scratch_types=[
            pltpu.VMEM((2, FDIM, VB), jnp.float32),
            pltpu.VMEM((vox_per_worker,), jnp.int32),
            pltpu.VMEM((NLANE * ACC_STRIDE,), jnp.float32),
            pltpu.VMEM((CNT_WORDS,), jnp.float32),
            pltpu.SemaphoreType.DMA,
            pltpu.SemaphoreType.DMA,
        ],
    )
    def sc_kern(emb_hbm, lbl_hbm, sums_hbm, cnts_hbm, ebuf, lbuf, acc, cnt,
                sem0, sem1):
        sems = (sem0, sem1)
        wid = lax.axis_index("s") * NCORE + lax.axis_index("c")
        b = wid // workers_per_batch
        vbase = (wid % workers_per_batch) * vox_per_worker

        zeros = jnp.zeros((NLANE,), jnp.float32)

        @pl.loop(0, NLANE * ACC_STRIDE // NLANE)
        def _zero(i):
            acc[pl.ds(i * NLANE, NLANE)] = zeros

        @pl.loop(0, CNT_WORDS // NLANE)
        def _zero_cnt(i):
            cnt[pl.ds(i * NLANE, NLANE)] = zeros

        pltpu.sync_copy(lbl_hbm.at[b, pl.ds(vbase, vox_per_worker)], lbuf)

        def tile_copy(blk, ib):
            return pltpu.make_async_copy(
                emb_hbm.at[b, :, pl.ds(vbase + blk * VB, VB)],
                ebuf.at[ib], sems[ib])

        tile_copy(0, 0).start()
        tile_copy(1, 1).start()

        lane = lax.iota(jnp.int32, NLANE)
        acc_lane_base = lane * ACC_STRIDE
        cnt_lane_base = lane * CNT_STRIDE
        ones = jnp.ones((NLANE,), jnp.float32)

        def process(ib, blk):
            tile_copy(blk, ib).wait()

            @pl.loop(0, VB // (2 * NLANE))
            def _grp(g):
                offs = [g * (2 * NLANE), g * (2 * NLANE) + NLANE]
                invs = []
                for off in offs:
                    parts = [jnp.zeros((NLANE,), jnp.float32)
                             for _ in range(4)]
                    for f in range(FDIM):
                        v = ebuf[ib, f, pl.ds(off, NLANE)]
                        parts[f % 4] = parts[f % 4] + v * v
                    ss = (parts[0] + parts[1]) + (parts[2] + parts[3])
                    invs.append(_newton_rsqrt(jnp.maximum(ss, 1e-24)))
                for off, inv in zip(offs, invs):
                    cls = lbuf[pl.ds(blk * VB + off, NLANE)]
                    plsc.addupdate_scatter(cnt, [cnt_lane_base + cls], ones)
                    acc_base = acc_lane_base + cls * FDIM
                    for f in range(FDIM):
                        v = ebuf[ib, f, pl.ds(off, NLANE)] * inv
                        plsc.addupdate_scatter(acc, [acc_base + f], v)

            @pl.when(blk + 2 < n_blocks)
            def _prefetch():
                tile_copy(blk + 2, ib).start()

        @pl.loop(0, n_blocks // 2)
        def _blocks(it):
            for ib in range(2):
                process(ib, it * 2 + ib)

        pltpu.sync_copy(acc, sums_hbm.at[wid])
        pltpu.sync_copy(cnt, cnts_hbm.at[wid])

    return sc_kern(emb, lbl)


def _tc_partial_body(emb_ref, lbl_ref, out_ref):
    i = pl.program_id(1)

    @pl.when(i == 0)
    def _init():
        out_ref[...] = jnp.zeros_like(out_ref)

    x = emb_ref[0]
    c = x.shape[-1]
    ss = jnp.sum(x * x, axis=0, keepdims=True)
    inv = lax.rsqrt(jnp.maximum(ss, 1e-24))
    xn = x * inv
    xe = jnp.concatenate(
        [xn, jnp.ones((1, c), jnp.float32),
         jnp.zeros((128 - FDIM - 1, c), jnp.float32)], axis=0)
    lab = lbl_ref[0, 0]
    oh = (lab == lax.broadcasted_iota(jnp.int32, (32, c), 0))
    part = lax.dot_general(oh.astype(jnp.float32), xe,
                           (((1,), (1,)), ((), ())),
                           preferred_element_type=jnp.float32)
    out_ref[0] += part


def _tc_segment_stats(emb, lbl, sc_n, n_vox):
    batch = emb.shape[0]
    chunk = 32768
    n_chunks = (n_vox - sc_n) // chunk
    first = sc_n // chunk
    lbl4 = lbl.reshape(batch, n_vox // chunk, 1, chunk)
    return pl.pallas_call(
        _tc_partial_body,
        grid=(batch, n_chunks),
        in_specs=[
            pl.BlockSpec((1, FDIM, chunk), lambda b, i: (b, 0, first + i)),
            pl.BlockSpec((1, 1, 1, chunk), lambda b, i: (b, first + i, 0, 0)),
        ],
        out_specs=pl.BlockSpec((1, 32, 128), lambda b, i: (b, 0, 0)),
        out_shape=jax.ShapeDtypeStruct((batch, 32, 128), jnp.float32),
    )(emb, lbl4)


def _finalize_body(sums_ref, cnts_ref, tc_ref, out_ref):
    rows_per_batch = sums_ref.shape[0] // 2
    s0 = jnp.sum(sums_ref[0:rows_per_batch], axis=0)
    s1 = jnp.sum(sums_ref[rows_per_batch:], axis=0)
    sums = jnp.stack([s0, s1])
    c0 = jnp.sum(cnts_ref[0:rows_per_batch], axis=0)
    c1 = jnp.sum(cnts_ref[rows_per_batch:], axis=0)
    counts = jnp.stack([c0, c1])[:, :NCLS]
    sums = sums + tc_ref[:, :NCLS, :FDIM]
    counts = counts + tc_ref[:, :NCLS, FDIM]

    mean_vector = sums / jnp.maximum(counts, 1.0)[:, :, None]
    mv_norm = jnp.sqrt(jnp.sum(mean_vector * mean_vector, axis=-1))
    R = jnp.clip(mv_norm, 1e-6, 0.999)
    mean_dir = mean_vector / jnp.maximum(mv_norm, 1e-12)[:, :, None]
    kappa = (R * FDIM - R * R * R) / (1.0 - R * R + 1e-6)
    is_bg = lax.broadcasted_iota(jnp.int32, (2, NCLS), 1) == 0
    kappa = jnp.where(is_bg, jnp.clip(kappa, 0.5, 10.0),
                      jnp.clip(kappa, 1.0, 500.0))
    valid = (counts >= 5.0).astype(jnp.float32)
    mean_dir = mean_dir * valid[:, :, None]
    kappa = kappa * valid
    Rv = R * valid
    out_ref[...] = jnp.concatenate(
        [mean_dir, kappa[:, :, None], Rv[:, :, None]], axis=-1)


def kernel(embeddings, labels):
    batch = embeddings.shape[0]
    n_vox = labels.shape[1] * labels.shape[2] * labels.shape[3]
    emb = embeddings.reshape(batch, FDIM, n_vox)
    lbl = labels.reshape(batch, n_vox)

    sc_n = SC_N
    sums, cnts = _sc_segment_stats(emb, lbl, sc_n)
    tc_part = _tc_segment_stats(emb, lbl, sc_n, n_vox)
    sums = sums.reshape(NWORK, NLANE, ACC_STRIDE)[:, :, :NCLS * FDIM]
    sums = sums.reshape(NWORK * NLANE, NCLS, FDIM)
    cnts = cnts[:, :NLANE * CNT_STRIDE].reshape(NWORK * NLANE, CNT_STRIDE)

    finalize = pl.pallas_call(
        _finalize_body,
        out_shape=jax.ShapeDtypeStruct((batch, NCLS, FDIM + 2), jnp.float32),
    )
    return finalize(sums, cnts, tc_part)


# speedup vs baseline: 1.0032x; 1.0032x over previous
import dataclasses
import functools

import jax
import jax.numpy as jnp
from jax import lax
from jax.experimental import pallas as pl
from jax.experimental.pallas import tpu as pltpu
from jax.experimental.pallas import tpu_sc as plsc

NCLS = 17
FDIM = 64
NLANE = 16
NCORE = 2
NSUB = 16
NWORK = NCORE * NSUB
VB = 512
ACC_STRIDE = NCLS * FDIM + 1
CNT_STRIDE = 33
CNT_WORDS = NLANE * 34
SC_N = 98304


def _newton_rsqrt(x):
    i = lax.bitcast_convert_type(x, jnp.int32)
    y = lax.bitcast_convert_type(jnp.int32(0x5F3759DF) - (i >> 1), jnp.float32)
    for _ in range(3):
        y = y * (1.5 - 0.5 * x * y * y)
    return y


def _sc_segment_stats(emb, lbl, sc_n):
    batch = emb.shape[0]
    vox_per_worker = batch * sc_n // NWORK
    workers_per_batch = NWORK // batch
    n_blocks = vox_per_worker // VB
    assert n_blocks >= 2 and n_blocks % 2 == 0 and n_blocks * VB == vox_per_worker

    mesh = plsc.VectorSubcoreMesh(core_axis_name="c", subcore_axis_name="s")
    cp = pltpu.CompilerParams()
    if "needs_layout_passes" in pltpu.CompilerParams.__dataclass_fields__:
        cp = dataclasses.replace(cp, needs_layout_passes=False)

    @functools.partial(
        pl.kernel,
        compiler_params=cp,
        out_type=(
            jax.ShapeDtypeStruct((NWORK, NLANE * ACC_STRIDE), jnp.float32),
            jax.ShapeDtypeStruct((NWORK, CNT_WORDS), jnp.float32),
        ),
        mesh=mesh,
        scratch_types=[
            pltpu.VMEM((2, FDIM, VB), jnp.float32),
            pltpu.VMEM((vox_per_worker,), jnp.int32),
            pltpu.VMEM((NLANE * ACC_STRIDE,), jnp.float32),
            pltpu.VMEM((CNT_WORDS,), jnp.float32),
            pltpu.SemaphoreType.DMA,
            pltpu.SemaphoreType.DMA,
        ],
    )
    def sc_kern(emb_hbm, lbl_hbm, sums_hbm, cnts_hbm, ebuf, lbuf, acc, cnt,
                sem0, sem1):
        sems = (sem0, sem1)
        wid = lax.axis_index("s") * NCORE + lax.axis_index("c")
        b = wid // workers_per_batch
        vbase = (wid % workers_per_batch) * vox_per_worker

        zeros = jnp.zeros((NLANE,), jnp.float32)

        @pl.loop(0, NLANE * ACC_STRIDE // NLANE)
        def _zero(i):
            acc[pl.ds(i * NLANE, NLANE)] = zeros

        @pl.loop(0, CNT_WORDS // NLANE)
        def _zero_cnt(i):
            cnt[pl.ds(i * NLANE, NLANE)] = zeros

        pltpu.sync_copy(lbl_hbm.at[b, pl.ds(vbase, vox_per_worker)], lbuf)

        def tile_copy(blk, ib):
            return pltpu.make_async_copy(
                emb_hbm.at[b, :, pl.ds(vbase + blk * VB, VB)],
                ebuf.at[ib], sems[ib])

        tile_copy(0, 0).start()
        tile_copy(1, 1).start()

        lane = lax.iota(jnp.int32, NLANE)
        acc_lane_base = lane * ACC_STRIDE
        cnt_lane_base = lane * CNT_STRIDE
        ones = jnp.ones((NLANE,), jnp.float32)

        def process(ib, blk):
            tile_copy(blk, ib).wait()

            @pl.loop(0, VB // (2 * NLANE))
            def _grp(g):
                offs = [g * (2 * NLANE), g * (2 * NLANE) + NLANE]
                invs = []
                for off in offs:
                    parts = [jnp.zeros((NLANE,), jnp.float32)
                             for _ in range(4)]
                    for f in range(FDIM):
                        v = ebuf[ib, f, pl.ds(off, NLANE)]
                        parts[f % 4] = parts[f % 4] + v * v
                    ss = (parts[0] + parts[1]) + (parts[2] + parts[3])
                    invs.append(_newton_rsqrt(jnp.maximum(ss, 1e-24)))
                for off, inv in zip(offs, invs):
                    cls = lbuf[pl.ds(blk * VB + off, NLANE)]
                    plsc.addupdate_scatter(cnt, [cnt_lane_base + cls], ones)
                    acc_base = acc_lane_base + cls * FDIM
                    for f in range(FDIM):
                        v = ebuf[ib, f, pl.ds(off, NLANE)] * inv
                        plsc.addupdate_scatter(acc, [acc_base + f], v)

            @pl.when(blk + 2 < n_blocks)
            def _prefetch():
                tile_copy(blk + 2, ib).start()

        @pl.loop(0, n_blocks // 2)
        def _blocks(it):
            for ib in range(2):
                process(ib, it * 2 + ib)

        pltpu.sync_copy(acc, sums_hbm.at[wid])
        pltpu.sync_copy(cnt, cnts_hbm.at[wid])

    return sc_kern(emb, lbl)


def _tc_partial_body(emb_ref, lbl_ref, out_ref):
    i = pl.program_id(1)

    @pl.when(i == 0)
    def _init():
        out_ref[...] = jnp.zeros_like(out_ref)

    x = emb_ref[0]
    c = x.shape[-1]
    ss = jnp.sum(x * x, axis=0, keepdims=True)
    inv = lax.rsqrt(jnp.maximum(ss, 1e-24))
    xn = x * inv
    xe = jnp.concatenate(
        [xn, jnp.ones((1, c), jnp.float32),
         jnp.zeros((128 - FDIM - 1, c), jnp.float32)], axis=0)
    lab = lbl_ref[0, 0]
    oh = (lab == lax.broadcasted_iota(jnp.int32, (32, c), 0))
    part = lax.dot_general(oh.astype(jnp.float32), xe,
                           (((1,), (1,)), ((), ())),
                           preferred_element_type=jnp.float32)
    out_ref[0] += part


def _tc_segment_stats(emb, lbl, sc_n, n_vox):
    batch = emb.shape[0]
    chunk = 16384
    n_chunks = (n_vox - sc_n) // chunk
    first = sc_n // chunk
    lbl4 = lbl.reshape(batch, n_vox // chunk, 1, chunk)
    return pl.pallas_call(
        _tc_partial_body,
        grid=(batch, n_chunks),
        in_specs=[
            pl.BlockSpec((1, FDIM, chunk), lambda b, i: (b, 0, first + i)),
            pl.BlockSpec((1, 1, 1, chunk), lambda b, i: (b, first + i, 0, 0)),
        ],
        out_specs=pl.BlockSpec((1, 32, 128), lambda b, i: (b, 0, 0)),
        out_shape=jax.ShapeDtypeStruct((batch, 32, 128), jnp.float32),
    )(emb, lbl4)


def _finalize_body(sums_ref, cnts_ref, tc_ref, out_ref):
    rows_per_batch = sums_ref.shape[0] // 2
    s0 = jnp.sum(sums_ref[0:rows_per_batch], axis=0)
    s1 = jnp.sum(sums_ref[rows_per_batch:], axis=0)
    sums = jnp.stack([s0, s1])
    c0 = jnp.sum(cnts_ref[0:rows_per_batch], axis=0)
    c1 = jnp.sum(cnts_ref[rows_per_batch:], axis=0)
    counts = jnp.stack([c0, c1])[:, :NCLS]
    sums = sums + tc_ref[:, :NCLS, :FDIM]
    counts = counts + tc_ref[:, :NCLS, FDIM]

    mean_vector = sums / jnp.maximum(counts, 1.0)[:, :, None]
    mv_norm = jnp.sqrt(jnp.sum(mean_vector * mean_vector, axis=-1))
    R = jnp.clip(mv_norm, 1e-6, 0.999)
    mean_dir = mean_vector / jnp.maximum(mv_norm, 1e-12)[:, :, None]
    kappa = (R * FDIM - R * R * R) / (1.0 - R * R + 1e-6)
    is_bg = lax.broadcasted_iota(jnp.int32, (2, NCLS), 1) == 0
    kappa = jnp.where(is_bg, jnp.clip(kappa, 0.5, 10.0),
                      jnp.clip(kappa, 1.0, 500.0))
    valid = (counts >= 5.0).astype(jnp.float32)
    mean_dir = mean_dir * valid[:, :, None]
    kappa = kappa * valid
    Rv = R * valid
    out_ref[...] = jnp.concatenate(
        [mean_dir, kappa[:, :, None], Rv[:, :, None]], axis=-1)


def kernel(embeddings, labels):
    batch = embeddings.shape[0]
    n_vox = labels.shape[1] * labels.shape[2] * labels.shape[3]
    emb = embeddings.reshape(batch, FDIM, n_vox)
    lbl = labels.reshape(batch, n_vox)

    sc_n = SC_N
    sums, cnts = _sc_segment_stats(emb, lbl, sc_n)
    tc_part = _tc_segment_stats(emb, lbl, sc_n, n_vox)
    sums = sums.reshape(NWORK, NLANE, ACC_STRIDE)[:, :, :NCLS * FDIM]
    sums = sums.reshape(NWORK * NLANE, NCLS, FDIM)
    cnts = cnts[:, :NLANE * CNT_STRIDE].reshape(NWORK * NLANE, CNT_STRIDE)

    finalize = pl.pallas_call(
        _finalize_body,
        out_shape=jax.ShapeDtypeStruct((batch, NCLS, FDIM + 2), jnp.float32),
    )
    return finalize(sums, cnts, tc_part)


# speedup vs baseline: 1.1344x; 1.1307x over previous
import dataclasses
import functools

import jax
import jax.numpy as jnp
from jax import lax
from jax.experimental import pallas as pl
from jax.experimental.pallas import tpu as pltpu
from jax.experimental.pallas import tpu_sc as plsc

NCLS = 17
FDIM = 64
NLANE = 16
NCORE = 2
NSUB = 16
NWORK = NCORE * NSUB
VB = 512
ACC_STRIDE = NCLS * FDIM + 1
CNT_STRIDE = 33
CNT_WORDS = NLANE * 34
SC_N = 65536


def _newton_rsqrt(x):
    i = lax.bitcast_convert_type(x, jnp.int32)
    y = lax.bitcast_convert_type(jnp.int32(0x5F3759DF) - (i >> 1), jnp.float32)
    for _ in range(3):
        y = y * (1.5 - 0.5 * x * y * y)
    return y


def _sc_segment_stats(emb, lbl, sc_n):
    batch = emb.shape[0]
    vox_per_worker = batch * sc_n // NWORK
    workers_per_batch = NWORK // batch
    n_blocks = vox_per_worker // VB
    assert n_blocks >= 2 and n_blocks % 2 == 0 and n_blocks * VB == vox_per_worker

    mesh = plsc.VectorSubcoreMesh(core_axis_name="c", subcore_axis_name="s")
    cp = pltpu.CompilerParams()
    if "needs_layout_passes" in pltpu.CompilerParams.__dataclass_fields__:
        cp = dataclasses.replace(cp, needs_layout_passes=False)

    @functools.partial(
        pl.kernel,
        compiler_params=cp,
        out_type=(
            jax.ShapeDtypeStruct((NWORK, NLANE * ACC_STRIDE), jnp.float32),
            jax.ShapeDtypeStruct((NWORK, CNT_WORDS), jnp.float32),
        ),
        mesh=mesh,
        scratch_types=[
            pltpu.VMEM((2, FDIM, VB), jnp.float32),
            pltpu.VMEM((vox_per_worker,), jnp.int32),
            pltpu.VMEM((NLANE * ACC_STRIDE,), jnp.float32),
            pltpu.VMEM((CNT_WORDS,), jnp.float32),
            pltpu.SemaphoreType.DMA,
            pltpu.SemaphoreType.DMA,
        ],
    )
    def sc_kern(emb_hbm, lbl_hbm, sums_hbm, cnts_hbm, ebuf, lbuf, acc, cnt,
                sem0, sem1):
        sems = (sem0, sem1)
        wid = lax.axis_index("s") * NCORE + lax.axis_index("c")
        b = wid // workers_per_batch
        vbase = (wid % workers_per_batch) * vox_per_worker

        zeros = jnp.zeros((NLANE,), jnp.float32)

        @pl.loop(0, NLANE * ACC_STRIDE // NLANE)
        def _zero(i):
            acc[pl.ds(i * NLANE, NLANE)] = zeros

        @pl.loop(0, CNT_WORDS // NLANE)
        def _zero_cnt(i):
            cnt[pl.ds(i * NLANE, NLANE)] = zeros

        pltpu.sync_copy(lbl_hbm.at[b, pl.ds(vbase, vox_per_worker)], lbuf)

        def tile_copy(blk, ib):
            return pltpu.make_async_copy(
                emb_hbm.at[b, :, pl.ds(vbase + blk * VB, VB)],
                ebuf.at[ib], sems[ib])

        tile_copy(0, 0).start()
        tile_copy(1, 1).start()

        lane = lax.iota(jnp.int32, NLANE)
        acc_lane_base = lane * ACC_STRIDE
        cnt_lane_base = lane * CNT_STRIDE
        ones = jnp.ones((NLANE,), jnp.float32)

        def process(ib, blk):
            tile_copy(blk, ib).wait()

            @pl.loop(0, VB // (2 * NLANE))
            def _grp(g):
                offs = [g * (2 * NLANE), g * (2 * NLANE) + NLANE]
                invs = []
                for off in offs:
                    parts = [jnp.zeros((NLANE,), jnp.float32)
                             for _ in range(4)]
                    for f in range(FDIM):
                        v = ebuf[ib, f, pl.ds(off, NLANE)]
                        parts[f % 4] = parts[f % 4] + v * v
                    ss = (parts[0] + parts[1]) + (parts[2] + parts[3])
                    invs.append(_newton_rsqrt(jnp.maximum(ss, 1e-24)))
                for off, inv in zip(offs, invs):
                    cls = lbuf[pl.ds(blk * VB + off, NLANE)]
                    plsc.addupdate_scatter(cnt, [cnt_lane_base + cls], ones)
                    acc_base = acc_lane_base + cls * FDIM
                    for f in range(FDIM):
                        v = ebuf[ib, f, pl.ds(off, NLANE)] * inv
                        plsc.addupdate_scatter(acc, [acc_base + f], v)

            @pl.when(blk + 2 < n_blocks)
            def _prefetch():
                tile_copy(blk + 2, ib).start()

        @pl.loop(0, n_blocks // 2)
        def _blocks(it):
            for ib in range(2):
                process(ib, it * 2 + ib)

        pltpu.sync_copy(acc, sums_hbm.at[wid])
        pltpu.sync_copy(cnt, cnts_hbm.at[wid])

    return sc_kern(emb, lbl)


def _tc_partial_body(emb_ref, lbl_ref, out_ref):
    i = pl.program_id(1)

    @pl.when(i == 0)
    def _init():
        out_ref[...] = jnp.zeros_like(out_ref)

    x = emb_ref[0]
    c = x.shape[-1]
    ss = jnp.sum(x * x, axis=0, keepdims=True)
    inv = lax.rsqrt(jnp.maximum(ss, 1e-24))
    xn = x * inv
    xe = jnp.concatenate(
        [xn, jnp.ones((1, c), jnp.float32),
         jnp.zeros((128 - FDIM - 1, c), jnp.float32)], axis=0)
    lab = lbl_ref[0, 0]
    oh = (lab == lax.broadcasted_iota(jnp.int32, (32, c), 0))
    part = lax.dot_general(oh.astype(jnp.float32), xe,
                           (((1,), (1,)), ((), ())),
                           preferred_element_type=jnp.float32)
    out_ref[0] += part


def _tc_segment_stats(emb, lbl, sc_n, n_vox):
    batch = emb.shape[0]
    chunk = 16384
    n_chunks = (n_vox - sc_n) // chunk
    first = sc_n // chunk
    lbl4 = lbl.reshape(batch, n_vox // chunk, 1, chunk)
    return pl.pallas_call(
        _tc_partial_body,
        grid=(batch, n_chunks),
        in_specs=[
            pl.BlockSpec((1, FDIM, chunk), lambda b, i: (b, 0, first + i)),
            pl.BlockSpec((1, 1, 1, chunk), lambda b, i: (b, first + i, 0, 0)),
        ],
        out_specs=pl.BlockSpec((1, 32, 128), lambda b, i: (b, 0, 0)),
        out_shape=jax.ShapeDtypeStruct((batch, 32, 128), jnp.float32),
    )(emb, lbl4)


def _finalize_body(sums_ref, cnts_ref, tc_ref, out_ref):
    rows_per_batch = sums_ref.shape[0] // 2
    s0 = jnp.sum(sums_ref[0:rows_per_batch], axis=0)
    s1 = jnp.sum(sums_ref[rows_per_batch:], axis=0)
    sums = jnp.stack([s0, s1])
    c0 = jnp.sum(cnts_ref[0:rows_per_batch], axis=0)
    c1 = jnp.sum(cnts_ref[rows_per_batch:], axis=0)
    counts = jnp.stack([c0, c1])[:, :NCLS]
    sums = sums + tc_ref[:, :NCLS, :FDIM]
    counts = counts + tc_ref[:, :NCLS, FDIM]

    mean_vector = sums / jnp.maximum(counts, 1.0)[:, :, None]
    mv_norm = jnp.sqrt(jnp.sum(mean_vector * mean_vector, axis=-1))
    R = jnp.clip(mv_norm, 1e-6, 0.999)
    mean_dir = mean_vector / jnp.maximum(mv_norm, 1e-12)[:, :, None]
    kappa = (R * FDIM - R * R * R) / (1.0 - R * R + 1e-6)
    is_bg = lax.broadcasted_iota(jnp.int32, (2, NCLS), 1) == 0
    kappa = jnp.where(is_bg, jnp.clip(kappa, 0.5, 10.0),
                      jnp.clip(kappa, 1.0, 500.0))
    valid = (counts >= 5.0).astype(jnp.float32)
    mean_dir = mean_dir * valid[:, :, None]
    kappa = kappa * valid
    Rv = R * valid
    out_ref[...] = jnp.concatenate(
        [mean_dir, kappa[:, :, None], Rv[:, :, None]], axis=-1)


def kernel(embeddings, labels):
    batch = embeddings.shape[0]
    n_vox = labels.shape[1] * labels.shape[2] * labels.shape[3]
    emb = embeddings.reshape(batch, FDIM, n_vox)
    lbl = labels.reshape(batch, n_vox)

    sc_n = SC_N
    sums, cnts = _sc_segment_stats(emb, lbl, sc_n)
    tc_part = _tc_segment_stats(emb, lbl, sc_n, n_vox)
    sums = sums.reshape(NWORK, NLANE, ACC_STRIDE)[:, :, :NCLS * FDIM]
    sums = sums.reshape(NWORK * NLANE, NCLS, FDIM)
    cnts = cnts[:, :NLANE * CNT_STRIDE].reshape(NWORK * NLANE, CNT_STRIDE)

    finalize = pl.pallas_call(
        _finalize_body,
        out_shape=jax.ShapeDtypeStruct((batch, NCLS, FDIM + 2), jnp.float32),
    )
    return finalize(sums, cnts, tc_part)


# speedup vs baseline: 1.2131x; 1.0694x over previous
import dataclasses
import functools

import jax
import jax.numpy as jnp
from jax import lax
from jax.experimental import pallas as pl
from jax.experimental.pallas import tpu as pltpu
from jax.experimental.pallas import tpu_sc as plsc

NCLS = 17
FDIM = 64
NLANE = 16
NCORE = 2
NSUB = 16
NWORK = NCORE * NSUB
VB = 512
ACC_STRIDE = NCLS * FDIM + 1
CNT_STRIDE = 33
CNT_WORDS = NLANE * 34
SC_N = 49152


def _newton_rsqrt(x):
    i = lax.bitcast_convert_type(x, jnp.int32)
    y = lax.bitcast_convert_type(jnp.int32(0x5F3759DF) - (i >> 1), jnp.float32)
    for _ in range(3):
        y = y * (1.5 - 0.5 * x * y * y)
    return y


def _sc_segment_stats(emb, lbl, sc_n):
    batch = emb.shape[0]
    vox_per_worker = batch * sc_n // NWORK
    workers_per_batch = NWORK // batch
    n_blocks = vox_per_worker // VB
    assert n_blocks >= 2 and n_blocks % 2 == 0 and n_blocks * VB == vox_per_worker

    mesh = plsc.VectorSubcoreMesh(core_axis_name="c", subcore_axis_name="s")
    cp = pltpu.CompilerParams()
    if "needs_layout_passes" in pltpu.CompilerParams.__dataclass_fields__:
        cp = dataclasses.replace(cp, needs_layout_passes=False)

    @functools.partial(
        pl.kernel,
        compiler_params=cp,
        out_type=(
            jax.ShapeDtypeStruct((NWORK, NLANE * ACC_STRIDE), jnp.float32),
            jax.ShapeDtypeStruct((NWORK, CNT_WORDS), jnp.float32),
        ),
        mesh=mesh,
        scratch_types=[
            pltpu.VMEM((2, FDIM, VB), jnp.float32),
            pltpu.VMEM((vox_per_worker,), jnp.int32),
            pltpu.VMEM((NLANE * ACC_STRIDE,), jnp.float32),
            pltpu.VMEM((CNT_WORDS,), jnp.float32),
            pltpu.SemaphoreType.DMA,
            pltpu.SemaphoreType.DMA,
        ],
    )
    def sc_kern(emb_hbm, lbl_hbm, sums_hbm, cnts_hbm, ebuf, lbuf, acc, cnt,
                sem0, sem1):
        sems = (sem0, sem1)
        wid = lax.axis_index("s") * NCORE + lax.axis_index("c")
        b = wid // workers_per_batch
        vbase = (wid % workers_per_batch) * vox_per_worker

        zeros = jnp.zeros((NLANE,), jnp.float32)

        @pl.loop(0, NLANE * ACC_STRIDE // NLANE)
        def _zero(i):
            acc[pl.ds(i * NLANE, NLANE)] = zeros

        @pl.loop(0, CNT_WORDS // NLANE)
        def _zero_cnt(i):
            cnt[pl.ds(i * NLANE, NLANE)] = zeros

        pltpu.sync_copy(lbl_hbm.at[b, pl.ds(vbase, vox_per_worker)], lbuf)

        def tile_copy(blk, ib):
            return pltpu.make_async_copy(
                emb_hbm.at[b, :, pl.ds(vbase + blk * VB, VB)],
                ebuf.at[ib], sems[ib])

        tile_copy(0, 0).start()
        tile_copy(1, 1).start()

        lane = lax.iota(jnp.int32, NLANE)
        acc_lane_base = lane * ACC_STRIDE
        cnt_lane_base = lane * CNT_STRIDE
        ones = jnp.ones((NLANE,), jnp.float32)

        def process(ib, blk):
            tile_copy(blk, ib).wait()

            @pl.loop(0, VB // (2 * NLANE))
            def _grp(g):
                offs = [g * (2 * NLANE), g * (2 * NLANE) + NLANE]
                invs = []
                for off in offs:
                    parts = [jnp.zeros((NLANE,), jnp.float32)
                             for _ in range(4)]
                    for f in range(FDIM):
                        v = ebuf[ib, f, pl.ds(off, NLANE)]
                        parts[f % 4] = parts[f % 4] + v * v
                    ss = (parts[0] + parts[1]) + (parts[2] + parts[3])
                    invs.append(_newton_rsqrt(jnp.maximum(ss, 1e-24)))
                for off, inv in zip(offs, invs):
                    cls = lbuf[pl.ds(blk * VB + off, NLANE)]
                    plsc.addupdate_scatter(cnt, [cnt_lane_base + cls], ones)
                    acc_base = acc_lane_base + cls * FDIM
                    for f in range(FDIM):
                        v = ebuf[ib, f, pl.ds(off, NLANE)] * inv
                        plsc.addupdate_scatter(acc, [acc_base + f], v)

            @pl.when(blk + 2 < n_blocks)
            def _prefetch():
                tile_copy(blk + 2, ib).start()

        @pl.loop(0, n_blocks // 2)
        def _blocks(it):
            for ib in range(2):
                process(ib, it * 2 + ib)

        pltpu.sync_copy(acc, sums_hbm.at[wid])
        pltpu.sync_copy(cnt, cnts_hbm.at[wid])

    return sc_kern(emb, lbl)


def _tc_partial_body(emb_ref, lbl_ref, out_ref):
    i = pl.program_id(1)

    @pl.when(i == 0)
    def _init():
        out_ref[...] = jnp.zeros_like(out_ref)

    x = emb_ref[0]
    c = x.shape[-1]
    ss = jnp.sum(x * x, axis=0, keepdims=True)
    inv = lax.rsqrt(jnp.maximum(ss, 1e-24))
    xn = x * inv
    xe = jnp.concatenate(
        [xn, jnp.ones((1, c), jnp.float32),
         jnp.zeros((128 - FDIM - 1, c), jnp.float32)], axis=0)
    lab = lbl_ref[0, 0]
    oh = (lab == lax.broadcasted_iota(jnp.int32, (32, c), 0))
    part = lax.dot_general(oh.astype(jnp.float32), xe,
                           (((1,), (1,)), ((), ())),
                           preferred_element_type=jnp.float32)
    out_ref[0] += part


def _tc_segment_stats(emb, lbl, sc_n, n_vox):
    batch = emb.shape[0]
    chunk = 16384
    n_chunks = (n_vox - sc_n) // chunk
    first = sc_n // chunk
    lbl4 = lbl.reshape(batch, n_vox // chunk, 1, chunk)
    return pl.pallas_call(
        _tc_partial_body,
        grid=(batch, n_chunks),
        in_specs=[
            pl.BlockSpec((1, FDIM, chunk), lambda b, i: (b, 0, first + i)),
            pl.BlockSpec((1, 1, 1, chunk), lambda b, i: (b, first + i, 0, 0)),
        ],
        out_specs=pl.BlockSpec((1, 32, 128), lambda b, i: (b, 0, 0)),
        out_shape=jax.ShapeDtypeStruct((batch, 32, 128), jnp.float32),
    )(emb, lbl4)


def _finalize_body(sums_ref, cnts_ref, tc_ref, out_ref):
    rows_per_batch = sums_ref.shape[0] // 2
    s0 = jnp.sum(sums_ref[0:rows_per_batch], axis=0)
    s1 = jnp.sum(sums_ref[rows_per_batch:], axis=0)
    sums = jnp.stack([s0, s1])
    c0 = jnp.sum(cnts_ref[0:rows_per_batch], axis=0)
    c1 = jnp.sum(cnts_ref[rows_per_batch:], axis=0)
    counts = jnp.stack([c0, c1])[:, :NCLS]
    sums = sums + tc_ref[:, :NCLS, :FDIM]
    counts = counts + tc_ref[:, :NCLS, FDIM]

    mean_vector = sums / jnp.maximum(counts, 1.0)[:, :, None]
    mv_norm = jnp.sqrt(jnp.sum(mean_vector * mean_vector, axis=-1))
    R = jnp.clip(mv_norm, 1e-6, 0.999)
    mean_dir = mean_vector / jnp.maximum(mv_norm, 1e-12)[:, :, None]
    kappa = (R * FDIM - R * R * R) / (1.0 - R * R + 1e-6)
    is_bg = lax.broadcasted_iota(jnp.int32, (2, NCLS), 1) == 0
    kappa = jnp.where(is_bg, jnp.clip(kappa, 0.5, 10.0),
                      jnp.clip(kappa, 1.0, 500.0))
    valid = (counts >= 5.0).astype(jnp.float32)
    mean_dir = mean_dir * valid[:, :, None]
    kappa = kappa * valid
    Rv = R * valid
    out_ref[...] = jnp.concatenate(
        [mean_dir, kappa[:, :, None], Rv[:, :, None]], axis=-1)


def kernel(embeddings, labels):
    batch = embeddings.shape[0]
    n_vox = labels.shape[1] * labels.shape[2] * labels.shape[3]
    emb = embeddings.reshape(batch, FDIM, n_vox)
    lbl = labels.reshape(batch, n_vox)

    sc_n = SC_N
    sums, cnts = _sc_segment_stats(emb, lbl, sc_n)
    tc_part = _tc_segment_stats(emb, lbl, sc_n, n_vox)
    sums = sums.reshape(NWORK, NLANE, ACC_STRIDE)[:, :, :NCLS * FDIM]
    sums = sums.reshape(NWORK * NLANE, NCLS, FDIM)
    cnts = cnts[:, :NLANE * CNT_STRIDE].reshape(NWORK * NLANE, CNT_STRIDE)

    finalize = pl.pallas_call(
        _finalize_body,
        out_shape=jax.ShapeDtypeStruct((batch, NCLS, FDIM + 2), jnp.float32),
    )
    return finalize(sums, cnts, tc_part)
